# baseline (speedup 1.0000x reference)
import dataclasses
import functools

import jax
import jax.numpy as jnp
from jax import lax
from jax.experimental import pallas as pl
from jax.experimental.pallas import tpu as pltpu
from jax.experimental.pallas import tpu_sc as plsc

B, T, C = 2, 2048, 2048
E, K, DFF = 8, 2, 2048
N = B * T
BM = 256
NBLK = (N * K) // BM + E
P = NBLK * BM
LANES = 128
CWI = 128
NCH = C // (2 * CWI)
SW = 128
NEG = -1e30


def _router_body(x_ref, wrt_ref, eidx_ref, rnk_ref, cnt_ref, cnt_s):
    b = pl.program_id(0)

    @pl.when(b == 0)
    def _():
        cnt_s[...] = jnp.zeros_like(cnt_s)

    lanes = lax.broadcasted_iota(jnp.int32, (BM, LANES), 1)
    logits = jnp.dot(x_ref[...], wrt_ref[...],
                     preferred_element_type=jnp.float32)
    neg = jnp.where(lanes < E, logits, NEG)
    m1 = jnp.max(neg, axis=1, keepdims=True)
    i1 = jnp.min(jnp.where(neg == m1, lanes, LANES - 1), axis=1, keepdims=True)
    oh1 = (lanes == i1)
    neg2 = jnp.where(oh1, NEG, neg)
    m2 = jnp.max(neg2, axis=1, keepdims=True)
    i2 = jnp.min(jnp.where(neg2 == m2, lanes, LANES - 1), axis=1, keepdims=True)
    oh2 = (lanes == i2)

    s = oh1.astype(jnp.float32) + oh2.astype(jnp.float32)
    rows = lax.broadcasted_iota(jnp.int32, (BM, BM), 0)
    cols = lax.broadcasted_iota(jnp.int32, (BM, BM), 1)
    lstrict = (rows > cols).astype(jnp.float32)
    pref = jnp.dot(lstrict, s, preferred_element_type=jnp.float32) + cnt_s[...]
    r0 = jnp.sum(pref * oh1, axis=1, keepdims=True)
    r1 = jnp.sum(pref * oh2, axis=1, keepdims=True)
    cnt_s[...] = cnt_s[...] + jnp.sum(s, axis=0, keepdims=True)

    eidx_ref[...] = jnp.concatenate([i1, i2], axis=1)
    rnk_ref[...] = jnp.concatenate([r0, r1], axis=1).astype(jnp.int32)
    cnt_ref[...] = cnt_s[...].astype(jnp.int32)


def _router(x_flat, wrt):
    return pl.pallas_call(
        _router_body,
        grid=(N // BM,),
        in_specs=[
            pl.BlockSpec((BM, C), lambda b: (b, 0)),
            pl.BlockSpec((C, LANES), lambda b: (0, 0)),
        ],
        out_specs=[
            pl.BlockSpec((BM, 2), lambda b: (b, 0)),
            pl.BlockSpec((BM, 2), lambda b: (b, 0)),
            pl.BlockSpec((1, LANES), lambda b: (0, 0)),
        ],
        out_shape=[
            jax.ShapeDtypeStruct((N, 2), jnp.int32),
            jax.ShapeDtypeStruct((N, 2), jnp.int32),
            jax.ShapeDtypeStruct((1, LANES), jnp.int32),
        ],
        scratch_shapes=[pltpu.VMEM((1, LANES), jnp.float32)],
    )(x_flat, wrt)


def _plan_body(eidx_ref, rnk_ref, cnt_ref, ci0_ref, ci1_ref, meta_ref):
    cnt = cnt_ref[...].astype(jnp.float32)
    pc = jnp.ceil(cnt * (1.0 / BM)) * BM
    rows = lax.broadcasted_iota(jnp.int32, (LANES, LANES), 0)
    cols = lax.broadcasted_iota(jnp.int32, (LANES, LANES), 1)
    lmask = (rows < cols).astype(jnp.float32)
    pbase = jnp.dot(pc, lmask, preferred_element_type=jnp.float32)
    cum = pbase + pc

    lanes = lax.broadcasted_iota(jnp.int32, (N, LANES), 1)
    e0 = eidx_ref[...][:, 0:1]
    e1 = eidx_ref[...][:, 1:2]
    b0 = jnp.sum(jnp.where(lanes == e0, pbase, 0.0), axis=1, keepdims=True)
    b1 = jnp.sum(jnp.where(lanes == e1, pbase, 0.0), axis=1, keepdims=True)
    pos0 = rnk_ref[...][:, 0:1] + b0.astype(jnp.int32)
    pos1 = rnk_ref[...][:, 1:2] + b1.astype(jnp.int32)
    ci = lax.broadcasted_iota(jnp.int32, (N, NCH), 1)
    ci0_ref[...] = pos0 * NCH + ci
    ci1_ref[...] = pos1 * NCH + ci

    bi = lax.broadcasted_iota(jnp.int32, (64, LANES), 0)
    bl = lax.broadcasted_iota(jnp.int32, (64, LANES), 1)
    ind = jnp.logical_and(cum <= (bi * BM).astype(jnp.float32), bl < E)
    be = jnp.minimum(jnp.sum(ind.astype(jnp.int32), axis=1, keepdims=True),
                     E - 1)
    nb = jnp.sum(pc * (1.0 / BM)).astype(jnp.int32)
    bi1 = lax.broadcasted_iota(jnp.int32, (64, 1), 0)
    meta_ref[...] = jnp.where(bi1 < NBLK, be,
                              jnp.where(bi1 == NBLK, nb, 0))


def _plan(eidx, rnk, cnt):
    return pl.pallas_call(
        _plan_body,
        grid=(1,),
        in_specs=[
            pl.BlockSpec((N, 2), lambda i: (0, 0)),
            pl.BlockSpec((N, 2), lambda i: (0, 0)),
            pl.BlockSpec((1, LANES), lambda i: (0, 0)),
        ],
        out_specs=[
            pl.BlockSpec((N, NCH), lambda i: (0, 0)),
            pl.BlockSpec((N, NCH), lambda i: (0, 0)),
            pl.BlockSpec((64, 1), lambda i: (0, 0)),
        ],
        out_shape=[
            jax.ShapeDtypeStruct((N, NCH), jnp.int32),
            jax.ShapeDtypeStruct((N, NCH), jnp.int32),
            jax.ShapeDtypeStruct((64, 1), jnp.int32),
        ],
    )(eidx, rnk, cnt)


def _dispatch(x_ch, ci0, ci1):
    mesh = plsc.VectorSubcoreMesh(core_axis_name="c", subcore_axis_name="s")

    @functools.partial(
        pl.kernel, mesh=mesh,
        out_type=jax.ShapeDtypeStruct((P * NCH, CWI), jnp.int32),
    )
    def k(x_hbm, i0_hbm, i1_hbm, xg_hbm):
        def body(x_vmem, i0_vmem, i1_vmem):
            pltpu.sync_copy(x_vmem, xg_hbm.at[i0_vmem.at[0]])
            pltpu.sync_copy(x_vmem, xg_hbm.at[i1_vmem.at[0]])

        pltpu.emit_pipeline(
            body,
            grid=(N * NCH // SW,),
            in_specs=[
                pl.BlockSpec((SW, CWI), lambda i: (i, 0)),
                pl.BlockSpec((1, SW), lambda i: (0, i)),
                pl.BlockSpec((1, SW), lambda i: (0, i)),
            ],
            out_specs=[],
            core_axis_name=("c", "s"),
            dimension_semantics=(pltpu.PARALLEL,),
        )(x_hbm, i0_hbm, i1_hbm)

    return k(x_ch, ci0, ci1)


def _mlp_body(meta_ref, xg_ref, w1_ref, w2_ref, wrt_ref, out_ref):
    b = pl.program_id(0)
    nb = meta_ref[NBLK]

    @pl.when(b >= nb)
    def _():
        out_ref[...] = jnp.zeros_like(out_ref)

    @pl.when(b < nb)
    def _():
        xbf = xg_ref[...]
        mid = lax.dot_general(xbf, w1_ref[0],
                              (((1,), (1,)), ((), ())),
                              preferred_element_type=jnp.float32)
        act = jnp.square(jnp.maximum(mid, 0.0)).astype(jnp.bfloat16)
        part = lax.dot_general(act, w2_ref[0],
                               (((1,), (1,)), ((), ())),
                               preferred_element_type=jnp.float32)

        lanes = lax.broadcasted_iota(jnp.int32, (BM, LANES), 1)
        logits = jnp.dot(xbf, wrt_ref[...], preferred_element_type=jnp.float32)
        negl = jnp.where(lanes < E, logits, NEG)
        m = jnp.max(negl, axis=1, keepdims=True)
        p = jnp.exp(negl - m)
        z = jnp.sum(p, axis=1, keepdims=True)
        e = meta_ref[b]
        w = jnp.sum(jnp.where(lanes == e, p, 0.0), axis=1, keepdims=True) / z
        out_ref[...] = (part * w).astype(jnp.bfloat16)


def _mlp(meta41, xg, w1, w2, wrt):
    grid_spec = pltpu.PrefetchScalarGridSpec(
        num_scalar_prefetch=1,
        grid=(NBLK,),
        in_specs=[
            pl.BlockSpec((BM, C), lambda b, m: (b, 0)),
            pl.BlockSpec((1, DFF, C), lambda b, m: (m[b], 0, 0)),
            pl.BlockSpec((1, C, DFF), lambda b, m: (m[b], 0, 0)),
            pl.BlockSpec((C, LANES), lambda b, m: (0, 0)),
        ],
        out_specs=pl.BlockSpec((BM, C), lambda b, m: (b, 0)),
    )
    return pl.pallas_call(
        _mlp_body,
        grid_spec=grid_spec,
        out_shape=jax.ShapeDtypeStruct((P, C), jnp.bfloat16),
    )(meta41, xg, w1, w2, wrt)


def _combine(os_ch, ci0, ci1):
    mesh = plsc.VectorSubcoreMesh(core_axis_name="c", subcore_axis_name="s")

    cp = pltpu.CompilerParams()
    if "needs_layout_passes" in pltpu.CompilerParams.__dataclass_fields__:
        cp = dataclasses.replace(cp, needs_layout_passes=False)

    @functools.partial(
        pl.kernel, mesh=mesh,
        out_type=jax.ShapeDtypeStruct((N * NCH, CWI), jnp.int32),
        compiler_params=cp,
        scratch_types=[pltpu.VMEM((SW, CWI), jnp.int32),
                       pltpu.SemaphoreType.DMA,
                       pltpu.SemaphoreType.DMA],
    )
    def k(os_hbm, i0_hbm, i1_hbm, fin_hbm, s1, sem0, sem1):
        def body(i0_vmem, i1_vmem, out_vmem):
            c0 = pltpu.async_copy(os_hbm.at[i0_vmem.at[0]], out_vmem, sem0)
            c1 = pltpu.async_copy(os_hbm.at[i1_vmem.at[0]], s1, sem1)
            c0.wait()
            c1.wait()

            @pl.loop(0, SW)
            def _(rr):
                for u in range(CWI // 16):
                    slc = (rr, pl.ds(u * 16, 16))
                    a = plsc.bitcast(out_vmem[slc], jnp.bfloat16)
                    b = plsc.bitcast(s1[slc], jnp.bfloat16)
                    out_vmem[slc] = plsc.bitcast(a + b, jnp.int32)

        pltpu.emit_pipeline(
            body,
            grid=(N * NCH // SW,),
            in_specs=[
                pl.BlockSpec((1, SW), lambda i: (0, i)),
                pl.BlockSpec((1, SW), lambda i: (0, i)),
            ],
            out_specs=[pl.BlockSpec((SW, CWI), lambda i: (i, 0))],
            core_axis_name=("c", "s"),
            dimension_semantics=(pltpu.PARALLEL,),
        )(i0_hbm, i1_hbm, fin_hbm)

    return k(os_ch, ci0, ci1)


def kernel(x, Wr, W1, W2):
    x_flat = x.reshape(-1, C)
    wrt = jnp.pad(Wr.T, ((0, 0), (0, LANES - E)))
    eidx, rnk, cnt = _router(x_flat, wrt)
    ci0, ci1, meta = _plan(eidx, rnk, cnt)
    meta41 = meta[:NBLK + 1, 0]
    ci0 = ci0.reshape(1, N * NCH)
    ci1 = ci1.reshape(1, N * NCH)
    xi = lax.bitcast_convert_type(
        x_flat.astype(jnp.bfloat16).reshape(N * NCH, CWI, 2), jnp.int32)
    xg = _dispatch(xi, ci0, ci1)
    xg_bf = lax.bitcast_convert_type(xg, jnp.bfloat16).reshape(P, C)
    out_slots = _mlp(meta41, xg_bf,
                     W1.astype(jnp.bfloat16), W2.astype(jnp.bfloat16),
                     wrt.astype(jnp.bfloat16))
    os_i = lax.bitcast_convert_type(
        out_slots.reshape(P * NCH, CWI, 2), jnp.int32)
    fin = _combine(os_i, ci0, ci1)
    fin_bf = lax.bitcast_convert_type(fin, jnp.bfloat16).reshape(N, C)
    return fin_bf.astype(jnp.float32).reshape(B, T, C)


# speedup vs baseline: 43.0380x; 43.0380x over previous
import functools

import jax
import jax.numpy as jnp
from jax import lax
from jax.experimental import pallas as pl
from jax.experimental.pallas import tpu as pltpu
from jax.experimental.pallas import tpu_sc as plsc

B, T, C = 2, 2048, 2048
E, K, DFF = 8, 2, 2048
N = B * T
BM = 256
NBLK = (N * K) // BM + E
P = NBLK * BM
LANES = 128
NS = C // LANES
NW = 32
TW = N // NW
NEG = -1e30


def _router_body(x_ref, wrt_ref, eidx_ref, rnk_ref, cnt_ref, *rest):
    xs_refs = rest[:NS]
    cnt_s = rest[NS]
    b = pl.program_id(0)

    @pl.when(b == 0)
    def _():
        cnt_s[...] = jnp.zeros_like(cnt_s)

    xb = x_ref[...]
    for j in range(NS):
        xs_refs[j][...] = xb[:, j * LANES:(j + 1) * LANES]

    lanes = lax.broadcasted_iota(jnp.int32, (BM, LANES), 1)
    logits = jnp.dot(xb, wrt_ref[...], preferred_element_type=jnp.float32)
    neg = jnp.where(lanes < E, logits, NEG)
    m1 = jnp.max(neg, axis=1, keepdims=True)
    i1 = jnp.min(jnp.where(neg == m1, lanes, LANES - 1), axis=1, keepdims=True)
    oh1 = (lanes == i1)
    neg2 = jnp.where(oh1, NEG, neg)
    m2 = jnp.max(neg2, axis=1, keepdims=True)
    i2 = jnp.min(jnp.where(neg2 == m2, lanes, LANES - 1), axis=1, keepdims=True)
    oh2 = (lanes == i2)

    s = oh1.astype(jnp.float32) + oh2.astype(jnp.float32)
    rows = lax.broadcasted_iota(jnp.int32, (BM, BM), 0)
    cols = lax.broadcasted_iota(jnp.int32, (BM, BM), 1)
    lstrict = (rows > cols).astype(jnp.float32)
    pref = jnp.dot(lstrict, s, preferred_element_type=jnp.float32) + cnt_s[...]
    r0 = jnp.sum(pref * oh1, axis=1, keepdims=True)
    r1 = jnp.sum(pref * oh2, axis=1, keepdims=True)
    cnt_s[...] = cnt_s[...] + jnp.sum(s, axis=0, keepdims=True)

    eidx_ref[...] = jnp.concatenate([i1, i2], axis=1)
    rnk_ref[...] = jnp.concatenate([r0, r1], axis=1).astype(jnp.int32)
    cnt_ref[...] = cnt_s[...].astype(jnp.int32)


def _router(x_flat, wrt):
    return pl.pallas_call(
        _router_body,
        grid=(N // BM,),
        in_specs=[
            pl.BlockSpec((BM, C), lambda b: (b, 0)),
            pl.BlockSpec((C, LANES), lambda b: (0, 0)),
        ],
        out_specs=[
            pl.BlockSpec((BM, 2), lambda b: (b, 0)),
            pl.BlockSpec((BM, 2), lambda b: (b, 0)),
            pl.BlockSpec((1, LANES), lambda b: (0, 0)),
        ] + [pl.BlockSpec((BM, LANES), lambda b: (b, 0)) for _ in range(NS)],
        out_shape=[
            jax.ShapeDtypeStruct((N, 2), jnp.int32),
            jax.ShapeDtypeStruct((N, 2), jnp.int32),
            jax.ShapeDtypeStruct((1, LANES), jnp.int32),
        ] + [jax.ShapeDtypeStruct((N, LANES), jnp.float32) for _ in range(NS)],
        scratch_shapes=[pltpu.VMEM((1, LANES), jnp.float32)],
    )(x_flat, wrt)


def _plan_body(eidx_ref, rnk_ref, cnt_ref, pos_ref, meta_ref):
    cnt = cnt_ref[...].astype(jnp.float32)
    pc = jnp.ceil(cnt * (1.0 / BM)) * BM
    rows = lax.broadcasted_iota(jnp.int32, (LANES, LANES), 0)
    cols = lax.broadcasted_iota(jnp.int32, (LANES, LANES), 1)
    lmask = (rows < cols).astype(jnp.float32)
    pbase = jnp.dot(pc, lmask, preferred_element_type=jnp.float32)
    cum = pbase + pc

    lanes = lax.broadcasted_iota(jnp.int32, (N, LANES), 1)
    e0 = eidx_ref[...][:, 0:1]
    e1 = eidx_ref[...][:, 1:2]
    b0 = jnp.sum(jnp.where(lanes == e0, pbase, 0.0), axis=1, keepdims=True)
    b1 = jnp.sum(jnp.where(lanes == e1, pbase, 0.0), axis=1, keepdims=True)
    pos0 = rnk_ref[...][:, 0:1] + b0.astype(jnp.int32)
    pos1 = rnk_ref[...][:, 1:2] + b1.astype(jnp.int32)
    pos_ref[...] = jnp.concatenate([pos0, pos1], axis=1)

    bi = lax.broadcasted_iota(jnp.int32, (64, LANES), 0)
    bl = lax.broadcasted_iota(jnp.int32, (64, LANES), 1)
    ind = jnp.logical_and(cum <= (bi * BM).astype(jnp.float32), bl < E)
    be = jnp.minimum(jnp.sum(ind.astype(jnp.int32), axis=1, keepdims=True),
                     E - 1)
    nb = jnp.sum(pc * (1.0 / BM)).astype(jnp.int32)
    bi1 = lax.broadcasted_iota(jnp.int32, (64, 1), 0)
    meta_ref[...] = jnp.where(bi1 < NBLK, be,
                              jnp.where(bi1 == NBLK, nb, 0))


def _plan(eidx, rnk, cnt):
    return pl.pallas_call(
        _plan_body,
        grid=(1,),
        in_specs=[
            pl.BlockSpec((N, 2), lambda i: (0, 0)),
            pl.BlockSpec((N, 2), lambda i: (0, 0)),
            pl.BlockSpec((1, LANES), lambda i: (0, 0)),
        ],
        out_specs=[
            pl.BlockSpec((N, 2), lambda i: (0, 0)),
            pl.BlockSpec((64, 1), lambda i: (0, 0)),
        ],
        out_shape=[
            jax.ShapeDtypeStruct((N, 2), jnp.int32),
            jax.ShapeDtypeStruct((64, 1), jnp.int32),
        ],
    )(eidx, rnk, cnt)


def _dispatch(xs, pos0, pos1):
    mesh = plsc.VectorSubcoreMesh(core_axis_name="c", subcore_axis_name="s")

    @functools.partial(
        pl.kernel, mesh=mesh,
        out_type=[jax.ShapeDtypeStruct((P, LANES), jnp.float32)
                  for _ in range(NS)],
        scratch_types=(
            [pltpu.VMEM((1, TW), jnp.int32) for _ in range(2)]
            + [pltpu.VMEM((TW, LANES), jnp.float32) for _ in range(2)]
            + [pltpu.SemaphoreType.DMA for _ in range(5)]
        ),
    )
    def k(*refs):
        xs_hbm = refs[:NS]
        i0_hbm, i1_hbm = refs[NS], refs[NS + 1]
        xg_hbm = refs[NS + 2:2 * NS + 2]
        i0v, i1v, vb0, vb1 = refs[2 * NS + 2:2 * NS + 6]
        isem, l0, l1, s0, s1 = refs[2 * NS + 6:]

        wid = lax.axis_index("s") * 2 + lax.axis_index("c")
        base = wid * TW
        pltpu.async_copy(i0_hbm.at[:, pl.ds(base, TW)], i0v, isem).wait()
        pltpu.async_copy(i1_hbm.at[:, pl.ds(base, TW)], i1v, isem).wait()

        bufs = (vb0, vb1)
        lsems = (l0, l1)
        ssems = (s0, s1)
        loads = [None, None]
        stores = [[], []]
        loads[0] = pltpu.async_copy(xs_hbm[0].at[pl.ds(base, TW)], vb0, l0)
        for j in range(NS):
            cur = j % 2
            nxt = (j + 1) % 2
            if j + 1 < NS:
                for h in stores[nxt]:
                    h.wait()
                stores[nxt] = []
                loads[nxt] = pltpu.async_copy(
                    xs_hbm[j + 1].at[pl.ds(base, TW)], bufs[nxt], lsems[nxt])
            loads[cur].wait()
            stores[cur].append(pltpu.async_copy(
                bufs[cur], xg_hbm[j].at[i0v.at[0]], ssems[cur]))
            stores[cur].append(pltpu.async_copy(
                bufs[cur], xg_hbm[j].at[i1v.at[0]], ssems[cur]))
        for side in stores:
            for h in side:
                h.wait()

    return k(*xs, pos0, pos1)


def _mlp_body(meta_ref, *refs):
    xg_refs = refs[:NS]
    w1_ref, w2_ref, wrt_ref = refs[NS:NS + 3]
    out_refs = refs[NS + 3:]
    b = pl.program_id(0)
    nb = meta_ref[NBLK]

    @pl.when(b >= nb)
    def _():
        for j in range(NS):
            out_refs[j][...] = jnp.zeros_like(out_refs[j])

    @pl.when(b < nb)
    def _():
        xgb = jnp.concatenate([r[...] for r in xg_refs], axis=1)
        xbf = xgb.astype(jnp.bfloat16)
        mid = lax.dot_general(xbf, w1_ref[0],
                              (((1,), (1,)), ((), ())),
                              preferred_element_type=jnp.float32)
        act = jnp.square(jnp.maximum(mid, 0.0)).astype(jnp.bfloat16)
        part = lax.dot_general(act, w2_ref[0],
                               (((1,), (1,)), ((), ())),
                               preferred_element_type=jnp.float32)

        lanes = lax.broadcasted_iota(jnp.int32, (BM, LANES), 1)
        logits = jnp.dot(xbf, wrt_ref[...], preferred_element_type=jnp.float32)
        negl = jnp.where(lanes < E, logits, NEG)
        m = jnp.max(negl, axis=1, keepdims=True)
        p = jnp.exp(negl - m)
        z = jnp.sum(p, axis=1, keepdims=True)
        e = meta_ref[b]
        w = jnp.sum(jnp.where(lanes == e, p, 0.0), axis=1, keepdims=True) / z
        pw = part * w
        for j in range(NS):
            out_refs[j][...] = pw[:, j * LANES:(j + 1) * LANES]


def _mlp(meta41, xgs, w1, w2, wrt):
    grid_spec = pltpu.PrefetchScalarGridSpec(
        num_scalar_prefetch=1,
        grid=(NBLK,),
        in_specs=(
            [pl.BlockSpec((BM, LANES), lambda b, m: (b, 0))
             for _ in range(NS)]
            + [
                pl.BlockSpec((1, DFF, C), lambda b, m: (m[b], 0, 0)),
                pl.BlockSpec((1, C, DFF), lambda b, m: (m[b], 0, 0)),
                pl.BlockSpec((C, LANES), lambda b, m: (0, 0)),
            ]
        ),
        out_specs=[pl.BlockSpec((BM, LANES), lambda b, m: (b, 0))
                   for _ in range(NS)],
    )
    return pl.pallas_call(
        _mlp_body,
        grid_spec=grid_spec,
        out_shape=[jax.ShapeDtypeStruct((P, LANES), jnp.float32)
                   for _ in range(NS)],
    )(meta41, *xgs, w1, w2, wrt)


def _combine(oss, pos0, pos1):
    mesh = plsc.VectorSubcoreMesh(core_axis_name="c", subcore_axis_name="s")

    @functools.partial(
        pl.kernel, mesh=mesh,
        out_type=[jax.ShapeDtypeStruct((N, LANES), jnp.float32)
                  for _ in range(NS)],
        scratch_types=(
            [pltpu.VMEM((1, TW), jnp.int32) for _ in range(2)]
            + [pltpu.VMEM((TW, LANES), jnp.float32) for _ in range(4)]
            + [pltpu.SemaphoreType.DMA for _ in range(5)]
        ),
    )
    def k(*refs):
        os_hbm = refs[:NS]
        i0_hbm, i1_hbm = refs[NS], refs[NS + 1]
        fin_hbm = refs[NS + 2:2 * NS + 2]
        i0v, i1v, ga0, gb0, ga1, gb1 = refs[2 * NS + 2:2 * NS + 8]
        isem, g0, g1, w0, w1 = refs[2 * NS + 8:]

        wid = lax.axis_index("s") * 2 + lax.axis_index("c")
        base = wid * TW
        pltpu.async_copy(i0_hbm.at[:, pl.ds(base, TW)], i0v, isem).wait()
        pltpu.async_copy(i1_hbm.at[:, pl.ds(base, TW)], i1v, isem).wait()

        abufs = (ga0, ga1)
        bbufs = (gb0, gb1)
        gsems = (g0, g1)
        wsems = (w0, w1)
        gets = [[], []]
        puts = [None, None]

        def start(j, side):
            gets[side].append(pltpu.async_copy(
                os_hbm[j].at[i0v.at[0]], abufs[side], gsems[side]))
            gets[side].append(pltpu.async_copy(
                os_hbm[j].at[i1v.at[0]], bbufs[side], gsems[side]))

        start(0, 0)
        for j in range(NS):
            cur = j % 2
            nxt = (j + 1) % 2
            if j + 1 < NS:
                if puts[nxt] is not None:
                    puts[nxt].wait()
                    puts[nxt] = None
                start(j + 1, nxt)
            for h in gets[cur]:
                h.wait()
            gets[cur] = []
            a, bb = abufs[cur], bbufs[cur]

            @pl.loop(0, TW)
            def _(rr):
                for u in range(LANES // 16):
                    slc = (rr, pl.ds(u * 16, 16))
                    a[slc] = a[slc] + bb[slc]

            puts[cur] = pltpu.async_copy(
                a, fin_hbm[j].at[pl.ds(base, TW)], wsems[cur])
        for h in puts:
            if h is not None:
                h.wait()

    return k(*oss, pos0, pos1)


def kernel(x, Wr, W1, W2):
    x_flat = x.reshape(-1, C)
    wrt = jnp.pad(Wr.T, ((0, 0), (0, LANES - E)))
    out = _router(x_flat, wrt)
    eidx, rnk, cnt = out[:3]
    xs = out[3:]
    posns, meta = _plan(eidx, rnk, cnt)
    meta41 = meta[:NBLK + 1, 0]
    pos0 = posns[:, 0].reshape(1, N)
    pos1 = posns[:, 1].reshape(1, N)
    xgs = _dispatch(xs, pos0, pos1)
    oss = _mlp(meta41, xgs,
               W1.astype(jnp.bfloat16), W2.astype(jnp.bfloat16),
               wrt.astype(jnp.bfloat16))
    fins = _combine(oss, pos0, pos1)
    return jnp.concatenate(fins, axis=1).reshape(B, T, C)


# speedup vs baseline: 43.1542x; 1.0027x over previous
import functools

import jax
import jax.numpy as jnp
from jax import lax
from jax.experimental import pallas as pl
from jax.experimental.pallas import tpu as pltpu
from jax.experimental.pallas import tpu_sc as plsc

B, T, C = 2, 2048, 2048
E, K, DFF = 8, 2, 2048
N = B * T
BM = 256
NBLK = (N * K) // BM + E
P = NBLK * BM
LANES = 128
NS = C // LANES
NW = 32
TW = N // NW
NEG = -1e30


def _router_body(x_ref, wrt_ref, eidx_ref, rnk_ref, cnt_ref, *rest):
    xs_refs = rest[:NS]
    cnt_s = rest[NS]
    b = pl.program_id(0)

    @pl.when(b == 0)
    def _():
        cnt_s[...] = jnp.zeros_like(cnt_s)

    xb = x_ref[...]
    for j in range(NS):
        xs_refs[j][...] = xb[:, j * LANES:(j + 1) * LANES]

    lanes = lax.broadcasted_iota(jnp.int32, (BM, LANES), 1)
    logits = jnp.dot(xb, wrt_ref[...], preferred_element_type=jnp.float32)
    neg = jnp.where(lanes < E, logits, NEG)
    m1 = jnp.max(neg, axis=1, keepdims=True)
    i1 = jnp.min(jnp.where(neg == m1, lanes, LANES - 1), axis=1, keepdims=True)
    oh1 = (lanes == i1)
    neg2 = jnp.where(oh1, NEG, neg)
    m2 = jnp.max(neg2, axis=1, keepdims=True)
    i2 = jnp.min(jnp.where(neg2 == m2, lanes, LANES - 1), axis=1, keepdims=True)
    oh2 = (lanes == i2)

    s = oh1.astype(jnp.float32) + oh2.astype(jnp.float32)
    rows = lax.broadcasted_iota(jnp.int32, (BM, BM), 0)
    cols = lax.broadcasted_iota(jnp.int32, (BM, BM), 1)
    lstrict = (rows > cols).astype(jnp.float32)
    pref = jnp.dot(lstrict, s, preferred_element_type=jnp.float32) + cnt_s[...]
    r0 = jnp.sum(pref * oh1, axis=1, keepdims=True)
    r1 = jnp.sum(pref * oh2, axis=1, keepdims=True)
    cnt_s[...] = cnt_s[...] + jnp.sum(s, axis=0, keepdims=True)

    eidx_ref[...] = jnp.concatenate([i1, i2], axis=1)
    rnk_ref[...] = jnp.concatenate([r0, r1], axis=1).astype(jnp.int32)
    cnt_ref[...] = cnt_s[...].astype(jnp.int32)


def _router(x_flat, wrt):
    return pl.pallas_call(
        _router_body,
        grid=(N // BM,),
        in_specs=[
            pl.BlockSpec((BM, C), lambda b: (b, 0)),
            pl.BlockSpec((C, LANES), lambda b: (0, 0)),
        ],
        out_specs=[
            pl.BlockSpec((BM, 2), lambda b: (b, 0)),
            pl.BlockSpec((BM, 2), lambda b: (b, 0)),
            pl.BlockSpec((1, LANES), lambda b: (0, 0)),
        ] + [pl.BlockSpec((BM, LANES), lambda b: (b, 0)) for _ in range(NS)],
        out_shape=[
            jax.ShapeDtypeStruct((N, 2), jnp.int32),
            jax.ShapeDtypeStruct((N, 2), jnp.int32),
            jax.ShapeDtypeStruct((1, LANES), jnp.int32),
        ] + [jax.ShapeDtypeStruct((N, LANES), jnp.float32) for _ in range(NS)],
        scratch_shapes=[pltpu.VMEM((1, LANES), jnp.float32)],
    )(x_flat, wrt)


def _plan_body(eidx_ref, rnk_ref, cnt_ref, pos_ref, meta_ref):
    cnt = cnt_ref[...].astype(jnp.float32)
    pc = jnp.ceil(cnt * (1.0 / BM)) * BM
    rows = lax.broadcasted_iota(jnp.int32, (LANES, LANES), 0)
    cols = lax.broadcasted_iota(jnp.int32, (LANES, LANES), 1)
    lmask = (rows < cols).astype(jnp.float32)
    pbase = jnp.dot(pc, lmask, preferred_element_type=jnp.float32)
    cum = pbase + pc

    lanes = lax.broadcasted_iota(jnp.int32, (N, LANES), 1)
    e0 = eidx_ref[...][:, 0:1]
    e1 = eidx_ref[...][:, 1:2]
    b0 = jnp.sum(jnp.where(lanes == e0, pbase, 0.0), axis=1, keepdims=True)
    b1 = jnp.sum(jnp.where(lanes == e1, pbase, 0.0), axis=1, keepdims=True)
    pos0 = rnk_ref[...][:, 0:1] + b0.astype(jnp.int32)
    pos1 = rnk_ref[...][:, 1:2] + b1.astype(jnp.int32)
    pos_ref[...] = jnp.concatenate([pos0, pos1], axis=1)

    bi = lax.broadcasted_iota(jnp.int32, (64, LANES), 0)
    bl = lax.broadcasted_iota(jnp.int32, (64, LANES), 1)
    ind = jnp.logical_and(cum <= (bi * BM).astype(jnp.float32), bl < E)
    be = jnp.minimum(jnp.sum(ind.astype(jnp.int32), axis=1, keepdims=True),
                     E - 1)
    nb = jnp.sum(pc * (1.0 / BM)).astype(jnp.int32)
    bi1 = lax.broadcasted_iota(jnp.int32, (64, 1), 0)
    meta_ref[...] = jnp.where(bi1 < NBLK, be,
                              jnp.where(bi1 == NBLK, nb, 0))


def _plan(eidx, rnk, cnt):
    return pl.pallas_call(
        _plan_body,
        grid=(1,),
        in_specs=[
            pl.BlockSpec((N, 2), lambda i: (0, 0)),
            pl.BlockSpec((N, 2), lambda i: (0, 0)),
            pl.BlockSpec((1, LANES), lambda i: (0, 0)),
        ],
        out_specs=[
            pl.BlockSpec((N, 2), lambda i: (0, 0)),
            pl.BlockSpec((64, 1), lambda i: (0, 0)),
        ],
        out_shape=[
            jax.ShapeDtypeStruct((N, 2), jnp.int32),
            jax.ShapeDtypeStruct((64, 1), jnp.int32),
        ],
    )(eidx, rnk, cnt)


def _dispatch(xs, pos0, pos1):
    mesh = plsc.VectorSubcoreMesh(core_axis_name="c", subcore_axis_name="s")

    @functools.partial(
        pl.kernel, mesh=mesh,
        out_type=[jax.ShapeDtypeStruct((P, LANES), jnp.float32)
                  for _ in range(NS)],
        scratch_types=(
            [pltpu.VMEM((1, TW), jnp.int32) for _ in range(2)]
            + [pltpu.VMEM((TW, LANES), jnp.float32) for _ in range(2)]
            + [pltpu.SemaphoreType.DMA for _ in range(5)]
        ),
    )
    def k(*refs):
        xs_hbm = refs[:NS]
        i0_hbm, i1_hbm = refs[NS], refs[NS + 1]
        xg_hbm = refs[NS + 2:2 * NS + 2]
        i0v, i1v, vb0, vb1 = refs[2 * NS + 2:2 * NS + 6]
        isem, l0, l1, s0, s1 = refs[2 * NS + 6:]

        wid = lax.axis_index("s") * 2 + lax.axis_index("c")
        base = wid * TW
        pltpu.async_copy(i0_hbm.at[:, pl.ds(base, TW)], i0v, isem).wait()
        pltpu.async_copy(i1_hbm.at[:, pl.ds(base, TW)], i1v, isem).wait()

        bufs = (vb0, vb1)
        lsems = (l0, l1)
        ssems = (s0, s1)
        loads = [None, None]
        stores = [[], []]
        loads[0] = pltpu.async_copy(xs_hbm[0].at[pl.ds(base, TW)], vb0, l0)
        for j in range(NS):
            cur = j % 2
            nxt = (j + 1) % 2
            if j + 1 < NS:
                for h in stores[nxt]:
                    h.wait()
                stores[nxt] = []
                loads[nxt] = pltpu.async_copy(
                    xs_hbm[j + 1].at[pl.ds(base, TW)], bufs[nxt], lsems[nxt])
            loads[cur].wait()
            stores[cur].append(pltpu.async_copy(
                bufs[cur], xg_hbm[j].at[i0v.at[0]], ssems[cur]))
            stores[cur].append(pltpu.async_copy(
                bufs[cur], xg_hbm[j].at[i1v.at[0]], ssems[cur]))
        for side in stores:
            for h in side:
                h.wait()

    return k(*xs, pos0, pos1)


def _cast_body(w1_ref, w2_ref, o1_ref, o2_ref):
    o1_ref[...] = w1_ref[...].astype(jnp.bfloat16)
    o2_ref[...] = w2_ref[...].astype(jnp.bfloat16)


def _cast_weights(w1, w2):
    return pl.pallas_call(
        _cast_body,
        grid=(E, 4),
        in_specs=[
            pl.BlockSpec((1, DFF // 4, C), lambda e, j: (e, j, 0)),
            pl.BlockSpec((1, C // 4, DFF), lambda e, j: (e, j, 0)),
        ],
        out_specs=[
            pl.BlockSpec((1, DFF // 4, C), lambda e, j: (e, j, 0)),
            pl.BlockSpec((1, C // 4, DFF), lambda e, j: (e, j, 0)),
        ],
        out_shape=[
            jax.ShapeDtypeStruct((E, DFF, C), jnp.bfloat16),
            jax.ShapeDtypeStruct((E, C, DFF), jnp.bfloat16),
        ],
    )(w1, w2)


def _mlp_body(meta_ref, *refs):
    xg_refs = refs[:NS]
    w1_ref, w2_ref, wrt_ref = refs[NS:NS + 3]
    out_refs = refs[NS + 3:]
    b = pl.program_id(0)
    nb = meta_ref[NBLK]

    @pl.when(b >= nb)
    def _():
        for j in range(NS):
            out_refs[j][...] = jnp.zeros_like(out_refs[j])

    @pl.when(b < nb)
    def _():
        xgb = jnp.concatenate([r[...] for r in xg_refs], axis=1)
        xbf = xgb.astype(jnp.bfloat16)
        mid = lax.dot_general(xbf, w1_ref[0],
                              (((1,), (1,)), ((), ())),
                              preferred_element_type=jnp.float32)
        act = jnp.square(jnp.maximum(mid, 0.0)).astype(jnp.bfloat16)
        part = lax.dot_general(act, w2_ref[0],
                               (((1,), (1,)), ((), ())),
                               preferred_element_type=jnp.float32)

        lanes = lax.broadcasted_iota(jnp.int32, (BM, LANES), 1)
        logits = jnp.dot(xbf, wrt_ref[...], preferred_element_type=jnp.float32)
        negl = jnp.where(lanes < E, logits, NEG)
        m = jnp.max(negl, axis=1, keepdims=True)
        p = jnp.exp(negl - m)
        z = jnp.sum(p, axis=1, keepdims=True)
        e = meta_ref[b]
        w = jnp.sum(jnp.where(lanes == e, p, 0.0), axis=1, keepdims=True) / z
        pw = part * w
        for j in range(NS):
            out_refs[j][...] = pw[:, j * LANES:(j + 1) * LANES]


def _mlp(meta41, xgs, w1, w2, wrt):
    grid_spec = pltpu.PrefetchScalarGridSpec(
        num_scalar_prefetch=1,
        grid=(NBLK,),
        in_specs=(
            [pl.BlockSpec((BM, LANES), lambda b, m: (b, 0))
             for _ in range(NS)]
            + [
                pl.BlockSpec((1, DFF, C), lambda b, m: (m[b], 0, 0)),
                pl.BlockSpec((1, C, DFF), lambda b, m: (m[b], 0, 0)),
                pl.BlockSpec((C, LANES), lambda b, m: (0, 0)),
            ]
        ),
        out_specs=[pl.BlockSpec((BM, LANES), lambda b, m: (b, 0))
                   for _ in range(NS)],
    )
    return pl.pallas_call(
        _mlp_body,
        grid_spec=grid_spec,
        out_shape=[jax.ShapeDtypeStruct((P, LANES), jnp.float32)
                   for _ in range(NS)],
    )(meta41, *xgs, w1, w2, wrt)


def _combine(oss, pos0, pos1):
    mesh = plsc.VectorSubcoreMesh(core_axis_name="c", subcore_axis_name="s")

    @functools.partial(
        pl.kernel, mesh=mesh,
        out_type=[jax.ShapeDtypeStruct((N, LANES), jnp.float32)
                  for _ in range(NS)],
        scratch_types=(
            [pltpu.VMEM((1, TW), jnp.int32) for _ in range(2)]
            + [pltpu.VMEM((TW, LANES), jnp.float32) for _ in range(4)]
            + [pltpu.SemaphoreType.DMA for _ in range(5)]
        ),
    )
    def k(*refs):
        os_hbm = refs[:NS]
        i0_hbm, i1_hbm = refs[NS], refs[NS + 1]
        fin_hbm = refs[NS + 2:2 * NS + 2]
        i0v, i1v, ga0, gb0, ga1, gb1 = refs[2 * NS + 2:2 * NS + 8]
        isem, g0, g1, w0, w1 = refs[2 * NS + 8:]

        wid = lax.axis_index("s") * 2 + lax.axis_index("c")
        base = wid * TW
        pltpu.async_copy(i0_hbm.at[:, pl.ds(base, TW)], i0v, isem).wait()
        pltpu.async_copy(i1_hbm.at[:, pl.ds(base, TW)], i1v, isem).wait()

        abufs = (ga0, ga1)
        bbufs = (gb0, gb1)
        gsems = (g0, g1)
        wsems = (w0, w1)
        gets = [[], []]
        puts = [None, None]

        def start(j, side):
            gets[side].append(pltpu.async_copy(
                os_hbm[j].at[i0v.at[0]], abufs[side], gsems[side]))
            gets[side].append(pltpu.async_copy(
                os_hbm[j].at[i1v.at[0]], bbufs[side], gsems[side]))

        start(0, 0)
        for j in range(NS):
            cur = j % 2
            nxt = (j + 1) % 2
            if j + 1 < NS:
                if puts[nxt] is not None:
                    puts[nxt].wait()
                    puts[nxt] = None
                start(j + 1, nxt)
            for h in gets[cur]:
                h.wait()
            gets[cur] = []
            a, bb = abufs[cur], bbufs[cur]

            @pl.loop(0, TW)
            def _(rr):
                for u in range(LANES // 16):
                    slc = (rr, pl.ds(u * 16, 16))
                    a[slc] = a[slc] + bb[slc]

            puts[cur] = pltpu.async_copy(
                a, fin_hbm[j].at[pl.ds(base, TW)], wsems[cur])
        for h in puts:
            if h is not None:
                h.wait()

    return k(*oss, pos0, pos1)


def kernel(x, Wr, W1, W2):
    x_flat = x.reshape(-1, C)
    wrt = jnp.pad(Wr.T, ((0, 0), (0, LANES - E)))
    out = _router(x_flat, wrt)
    eidx, rnk, cnt = out[:3]
    xs = out[3:]
    posns, meta = _plan(eidx, rnk, cnt)
    meta41 = meta[:NBLK + 1, 0]
    pos0 = posns[:, 0].reshape(1, N)
    pos1 = posns[:, 1].reshape(1, N)
    xgs = _dispatch(xs, pos0, pos1)
    w1b, w2b = _cast_weights(W1, W2)
    oss = _mlp(meta41, xgs, w1b, w2b, wrt.astype(jnp.bfloat16))
    fins = _combine(oss, pos0, pos1)
    return jnp.concatenate(fins, axis=1).reshape(B, T, C)


# speedup vs baseline: 43.9630x; 1.0187x over previous
import functools

import jax
import jax.numpy as jnp
from jax import lax
from jax.experimental import pallas as pl
from jax.experimental.pallas import tpu as pltpu
from jax.experimental.pallas import tpu_sc as plsc

B, T, C = 2, 2048, 2048
E, K, DFF = 8, 2, 2048
N = B * T
BM = 512
NBLK = (N * K) // BM + E
P = NBLK * BM
LANES = 128
NS = C // LANES
NW = 32
TW = N // NW
NEG = -1e30


def _router_body(x_ref, wrt_ref, eidx_ref, rnk_ref, cnt_ref, *rest):
    xs_refs = rest[:NS]
    cnt_s = rest[NS]
    b = pl.program_id(0)

    @pl.when(b == 0)
    def _():
        cnt_s[...] = jnp.zeros_like(cnt_s)

    xb = x_ref[...]
    for j in range(NS):
        xs_refs[j][...] = xb[:, j * LANES:(j + 1) * LANES]

    lanes = lax.broadcasted_iota(jnp.int32, (BM, LANES), 1)
    logits = jnp.dot(xb, wrt_ref[...], preferred_element_type=jnp.float32)
    neg = jnp.where(lanes < E, logits, NEG)
    m1 = jnp.max(neg, axis=1, keepdims=True)
    i1 = jnp.min(jnp.where(neg == m1, lanes, LANES - 1), axis=1, keepdims=True)
    oh1 = (lanes == i1)
    neg2 = jnp.where(oh1, NEG, neg)
    m2 = jnp.max(neg2, axis=1, keepdims=True)
    i2 = jnp.min(jnp.where(neg2 == m2, lanes, LANES - 1), axis=1, keepdims=True)
    oh2 = (lanes == i2)

    s = oh1.astype(jnp.float32) + oh2.astype(jnp.float32)
    rows = lax.broadcasted_iota(jnp.int32, (BM, BM), 0)
    cols = lax.broadcasted_iota(jnp.int32, (BM, BM), 1)
    lstrict = (rows > cols).astype(jnp.float32)
    pref = jnp.dot(lstrict, s, preferred_element_type=jnp.float32) + cnt_s[...]
    r0 = jnp.sum(pref * oh1, axis=1, keepdims=True)
    r1 = jnp.sum(pref * oh2, axis=1, keepdims=True)
    cnt_s[...] = cnt_s[...] + jnp.sum(s, axis=0, keepdims=True)

    eidx_ref[...] = jnp.concatenate([i1, i2], axis=1)
    rnk_ref[...] = jnp.concatenate([r0, r1], axis=1).astype(jnp.int32)
    cnt_ref[...] = cnt_s[...].astype(jnp.int32)


def _router(x_flat, wrt):
    return pl.pallas_call(
        _router_body,
        grid=(N // BM,),
        in_specs=[
            pl.BlockSpec((BM, C), lambda b: (b, 0)),
            pl.BlockSpec((C, LANES), lambda b: (0, 0)),
        ],
        out_specs=[
            pl.BlockSpec((BM, 2), lambda b: (b, 0)),
            pl.BlockSpec((BM, 2), lambda b: (b, 0)),
            pl.BlockSpec((1, LANES), lambda b: (0, 0)),
        ] + [pl.BlockSpec((BM, LANES), lambda b: (b, 0)) for _ in range(NS)],
        out_shape=[
            jax.ShapeDtypeStruct((N, 2), jnp.int32),
            jax.ShapeDtypeStruct((N, 2), jnp.int32),
            jax.ShapeDtypeStruct((1, LANES), jnp.int32),
        ] + [jax.ShapeDtypeStruct((N, LANES), jnp.float32) for _ in range(NS)],
        scratch_shapes=[pltpu.VMEM((1, LANES), jnp.float32)],
    )(x_flat, wrt)


def _plan_body(eidx_ref, rnk_ref, cnt_ref, pos_ref, meta_ref):
    cnt = cnt_ref[...].astype(jnp.float32)
    pc = jnp.ceil(cnt * (1.0 / BM)) * BM
    rows = lax.broadcasted_iota(jnp.int32, (LANES, LANES), 0)
    cols = lax.broadcasted_iota(jnp.int32, (LANES, LANES), 1)
    lmask = (rows < cols).astype(jnp.float32)
    pbase = jnp.dot(pc, lmask, preferred_element_type=jnp.float32)
    cum = pbase + pc

    lanes = lax.broadcasted_iota(jnp.int32, (N, LANES), 1)
    e0 = eidx_ref[...][:, 0:1]
    e1 = eidx_ref[...][:, 1:2]
    b0 = jnp.sum(jnp.where(lanes == e0, pbase, 0.0), axis=1, keepdims=True)
    b1 = jnp.sum(jnp.where(lanes == e1, pbase, 0.0), axis=1, keepdims=True)
    pos0 = rnk_ref[...][:, 0:1] + b0.astype(jnp.int32)
    pos1 = rnk_ref[...][:, 1:2] + b1.astype(jnp.int32)
    pos_ref[...] = jnp.concatenate([pos0, pos1], axis=1)

    bi = lax.broadcasted_iota(jnp.int32, (64, LANES), 0)
    bl = lax.broadcasted_iota(jnp.int32, (64, LANES), 1)
    ind = jnp.logical_and(cum <= (bi * BM).astype(jnp.float32), bl < E)
    be = jnp.minimum(jnp.sum(ind.astype(jnp.int32), axis=1, keepdims=True),
                     E - 1)
    nb = jnp.sum(pc * (1.0 / BM)).astype(jnp.int32)
    bi1 = lax.broadcasted_iota(jnp.int32, (64, 1), 0)
    meta_ref[...] = jnp.where(bi1 < NBLK, be,
                              jnp.where(bi1 == NBLK, nb, 0))


def _plan(eidx, rnk, cnt):
    return pl.pallas_call(
        _plan_body,
        grid=(1,),
        in_specs=[
            pl.BlockSpec((N, 2), lambda i: (0, 0)),
            pl.BlockSpec((N, 2), lambda i: (0, 0)),
            pl.BlockSpec((1, LANES), lambda i: (0, 0)),
        ],
        out_specs=[
            pl.BlockSpec((N, 2), lambda i: (0, 0)),
            pl.BlockSpec((64, 1), lambda i: (0, 0)),
        ],
        out_shape=[
            jax.ShapeDtypeStruct((N, 2), jnp.int32),
            jax.ShapeDtypeStruct((64, 1), jnp.int32),
        ],
    )(eidx, rnk, cnt)


def _dispatch(xs, pos0, pos1):
    mesh = plsc.VectorSubcoreMesh(core_axis_name="c", subcore_axis_name="s")

    @functools.partial(
        pl.kernel, mesh=mesh,
        out_type=[jax.ShapeDtypeStruct((P, LANES), jnp.float32)
                  for _ in range(NS)],
        scratch_types=(
            [pltpu.VMEM((1, TW), jnp.int32) for _ in range(2)]
            + [pltpu.VMEM((TW, LANES), jnp.float32) for _ in range(2)]
            + [pltpu.SemaphoreType.DMA for _ in range(5)]
        ),
    )
    def k(*refs):
        xs_hbm = refs[:NS]
        i0_hbm, i1_hbm = refs[NS], refs[NS + 1]
        xg_hbm = refs[NS + 2:2 * NS + 2]
        i0v, i1v, vb0, vb1 = refs[2 * NS + 2:2 * NS + 6]
        isem, l0, l1, s0, s1 = refs[2 * NS + 6:]

        wid = lax.axis_index("s") * 2 + lax.axis_index("c")
        base = wid * TW
        pltpu.async_copy(i0_hbm.at[:, pl.ds(base, TW)], i0v, isem).wait()
        pltpu.async_copy(i1_hbm.at[:, pl.ds(base, TW)], i1v, isem).wait()

        bufs = (vb0, vb1)
        lsems = (l0, l1)
        ssems = (s0, s1)
        loads = [None, None]
        stores = [[], []]
        loads[0] = pltpu.async_copy(xs_hbm[0].at[pl.ds(base, TW)], vb0, l0)
        for j in range(NS):
            cur = j % 2
            nxt = (j + 1) % 2
            if j + 1 < NS:
                for h in stores[nxt]:
                    h.wait()
                stores[nxt] = []
                loads[nxt] = pltpu.async_copy(
                    xs_hbm[j + 1].at[pl.ds(base, TW)], bufs[nxt], lsems[nxt])
            loads[cur].wait()
            stores[cur].append(pltpu.async_copy(
                bufs[cur], xg_hbm[j].at[i0v.at[0]], ssems[cur]))
            stores[cur].append(pltpu.async_copy(
                bufs[cur], xg_hbm[j].at[i1v.at[0]], ssems[cur]))
        for side in stores:
            for h in side:
                h.wait()

    return k(*xs, pos0, pos1)


def _cast_body(w1_ref, w2_ref, o1_ref, o2_ref):
    o1_ref[...] = w1_ref[...].astype(jnp.bfloat16)
    o2_ref[...] = w2_ref[...].astype(jnp.bfloat16)


def _cast_weights(w1, w2):
    return pl.pallas_call(
        _cast_body,
        grid=(E, 4),
        in_specs=[
            pl.BlockSpec((1, DFF // 4, C), lambda e, j: (e, j, 0)),
            pl.BlockSpec((1, C // 4, DFF), lambda e, j: (e, j, 0)),
        ],
        out_specs=[
            pl.BlockSpec((1, DFF // 4, C), lambda e, j: (e, j, 0)),
            pl.BlockSpec((1, C // 4, DFF), lambda e, j: (e, j, 0)),
        ],
        out_shape=[
            jax.ShapeDtypeStruct((E, DFF, C), jnp.bfloat16),
            jax.ShapeDtypeStruct((E, C, DFF), jnp.bfloat16),
        ],
    )(w1, w2)


def _mlp_body(meta_ref, *refs):
    xg_refs = refs[:NS]
    w1_ref, w2_ref, wrt_ref = refs[NS:NS + 3]
    out_refs = refs[NS + 3:]
    b = pl.program_id(0)
    nb = meta_ref[NBLK]

    @pl.when(b >= nb)
    def _():
        for j in range(NS):
            out_refs[j][...] = jnp.zeros_like(out_refs[j])

    @pl.when(b < nb)
    def _():
        xgb = jnp.concatenate([r[...] for r in xg_refs], axis=1)
        xbf = xgb.astype(jnp.bfloat16)
        mid = lax.dot_general(xbf, w1_ref[0],
                              (((1,), (1,)), ((), ())),
                              preferred_element_type=jnp.float32)
        act = jnp.square(jnp.maximum(mid, 0.0)).astype(jnp.bfloat16)
        part = lax.dot_general(act, w2_ref[0],
                               (((1,), (1,)), ((), ())),
                               preferred_element_type=jnp.float32)

        lanes = lax.broadcasted_iota(jnp.int32, (BM, LANES), 1)
        logits = jnp.dot(xbf, wrt_ref[...], preferred_element_type=jnp.float32)
        negl = jnp.where(lanes < E, logits, NEG)
        m = jnp.max(negl, axis=1, keepdims=True)
        p = jnp.exp(negl - m)
        z = jnp.sum(p, axis=1, keepdims=True)
        e = meta_ref[b]
        w = jnp.sum(jnp.where(lanes == e, p, 0.0), axis=1, keepdims=True) / z
        pw = part * w
        for j in range(NS):
            out_refs[j][...] = pw[:, j * LANES:(j + 1) * LANES]


def _mlp(meta41, xgs, w1, w2, wrt):
    grid_spec = pltpu.PrefetchScalarGridSpec(
        num_scalar_prefetch=1,
        grid=(NBLK,),
        in_specs=(
            [pl.BlockSpec((BM, LANES), lambda b, m: (b, 0))
             for _ in range(NS)]
            + [
                pl.BlockSpec((1, DFF, C), lambda b, m: (m[b], 0, 0)),
                pl.BlockSpec((1, C, DFF), lambda b, m: (m[b], 0, 0)),
                pl.BlockSpec((C, LANES), lambda b, m: (0, 0)),
            ]
        ),
        out_specs=[pl.BlockSpec((BM, LANES), lambda b, m: (b, 0))
                   for _ in range(NS)],
    )
    return pl.pallas_call(
        _mlp_body,
        grid_spec=grid_spec,
        out_shape=[jax.ShapeDtypeStruct((P, LANES), jnp.float32)
                   for _ in range(NS)],
    )(meta41, *xgs, w1, w2, wrt)


def _combine(oss, pos0, pos1):
    mesh = plsc.VectorSubcoreMesh(core_axis_name="c", subcore_axis_name="s")

    @functools.partial(
        pl.kernel, mesh=mesh,
        out_type=[jax.ShapeDtypeStruct((N, LANES), jnp.float32)
                  for _ in range(NS)],
        scratch_types=(
            [pltpu.VMEM((1, TW), jnp.int32) for _ in range(2)]
            + [pltpu.VMEM((TW, LANES), jnp.float32) for _ in range(4)]
            + [pltpu.SemaphoreType.DMA for _ in range(5)]
        ),
    )
    def k(*refs):
        os_hbm = refs[:NS]
        i0_hbm, i1_hbm = refs[NS], refs[NS + 1]
        fin_hbm = refs[NS + 2:2 * NS + 2]
        i0v, i1v, ga0, gb0, ga1, gb1 = refs[2 * NS + 2:2 * NS + 8]
        isem, g0, g1, w0, w1 = refs[2 * NS + 8:]

        wid = lax.axis_index("s") * 2 + lax.axis_index("c")
        base = wid * TW
        pltpu.async_copy(i0_hbm.at[:, pl.ds(base, TW)], i0v, isem).wait()
        pltpu.async_copy(i1_hbm.at[:, pl.ds(base, TW)], i1v, isem).wait()

        abufs = (ga0, ga1)
        bbufs = (gb0, gb1)
        gsems = (g0, g1)
        wsems = (w0, w1)
        gets = [[], []]
        puts = [None, None]

        def start(j, side):
            gets[side].append(pltpu.async_copy(
                os_hbm[j].at[i0v.at[0]], abufs[side], gsems[side]))
            gets[side].append(pltpu.async_copy(
                os_hbm[j].at[i1v.at[0]], bbufs[side], gsems[side]))

        start(0, 0)
        for j in range(NS):
            cur = j % 2
            nxt = (j + 1) % 2
            if j + 1 < NS:
                if puts[nxt] is not None:
                    puts[nxt].wait()
                    puts[nxt] = None
                start(j + 1, nxt)
            for h in gets[cur]:
                h.wait()
            gets[cur] = []
            a, bb = abufs[cur], bbufs[cur]

            @pl.loop(0, TW)
            def _(rr):
                for u in range(LANES // 16):
                    slc = (rr, pl.ds(u * 16, 16))
                    a[slc] = a[slc] + bb[slc]

            puts[cur] = pltpu.async_copy(
                a, fin_hbm[j].at[pl.ds(base, TW)], wsems[cur])
        for h in puts:
            if h is not None:
                h.wait()

    return k(*oss, pos0, pos1)


def kernel(x, Wr, W1, W2):
    x_flat = x.reshape(-1, C)
    wrt = jnp.pad(Wr.T, ((0, 0), (0, LANES - E)))
    out = _router(x_flat, wrt)
    eidx, rnk, cnt = out[:3]
    xs = out[3:]
    posns, meta = _plan(eidx, rnk, cnt)
    meta41 = meta[:NBLK + 1, 0]
    pos0 = posns[:, 0].reshape(1, N)
    pos1 = posns[:, 1].reshape(1, N)
    xgs = _dispatch(xs, pos0, pos1)
    w1b, w2b = _cast_weights(W1, W2)
    oss = _mlp(meta41, xgs, w1b, w2b, wrt.astype(jnp.bfloat16))
    fins = _combine(oss, pos0, pos1)
    return jnp.concatenate(fins, axis=1).reshape(B, T, C)
